# baseline (speedup 1.0000x reference)
import functools

import jax
import jax.numpy as jnp
from jax import lax
from jax.experimental import pallas as pl
from jax.experimental.pallas import tpu as pltpu
from jax.experimental.pallas import tpu_sc as plsc

D_MODEL = 64
EPS = 1e-05

_GATHER_DNUMS = lax.GatherDimensionNumbers(
    offset_dims=(), collapsed_slice_dims=(0,), start_index_map=(0,))


def _lane_perm(x, idx):
    return lax.gather(x, idx[:, None], _GATHER_DNUMS, slice_sizes=(1,),
                      mode=lax.GatherScatterMode.PROMISE_IN_BOUNDS)


def _lane_allsum(x):
    lanes = lax.iota(jnp.int32, 16)
    for sh in (8, 4, 2, 1):
        x = x + _lane_perm(x, lanes ^ sh)
    return x

NUM_CORES = 2
NUM_SUBCORES = 16
NUM_WORKERS = NUM_CORES * NUM_SUBCORES

GATHER = 128
GATHERS_PER_STEP = 8
CHUNK = GATHER * GATHERS_PER_STEP


def _sc_embed_logmap(idx2d, table, n_rows):
    rows_per_w = n_rows // NUM_WORKERS
    steps = rows_per_w // CHUNK
    mesh = plsc.VectorSubcoreMesh(core_axis_name="c", subcore_axis_name="s")

    @functools.partial(
        pl.kernel,
        mesh=mesh,
        out_type=jax.ShapeDtypeStruct((n_rows, D_MODEL), jnp.float32),
        scratch_types=[
            pltpu.VMEM((GATHERS_PER_STEP, GATHER), jnp.int32),
            pltpu.VMEM((CHUNK, D_MODEL), jnp.float32),
            pltpu.SemaphoreType.DMA,
        ],
        compiler_params=pltpu.CompilerParams(use_tc_tiling_on_sc=False),
    )
    def body(idx_hbm, table_hbm, out_hbm, idx_v, rows_v, sem):
        wid = lax.axis_index("s") * NUM_CORES + lax.axis_index("c")
        row_base = wid * rows_per_w
        gather_row_base = row_base // GATHER

        def step(g, carry):
            row_off = pl.multiple_of(row_base + g * CHUNK, CHUNK)
            idx_off = pl.multiple_of(
                gather_row_base + g * GATHERS_PER_STEP, GATHERS_PER_STEP)
            pltpu.sync_copy(
                idx_hbm.at[pl.ds(idx_off, GATHERS_PER_STEP)],
                idx_v,
            )
            copies = []
            for b in range(GATHERS_PER_STEP):
                copies.append(pltpu.async_copy(
                    table_hbm.at[idx_v.at[b]],
                    rows_v.at[pl.ds(b * GATHER, GATHER)],
                    sem,
                ))
            for c in copies:
                c.wait()

            def row_fix(i, c):
                r0 = rows_v[i, pl.ds(0, 16)]
                r1 = rows_v[i, pl.ds(16, 16)]
                r2 = rows_v[i, pl.ds(32, 16)]
                r3 = rows_v[i, pl.ds(48, 16)]
                s = r0 * r0 + r1 * r1 + r2 * r2 + r3 * r3
                nsv = _lane_allsum(s)
                scale = 2.0 / ((1.0 + EPS) - nsv)
                rows_v[i, pl.ds(0, 16)] = r0 * scale
                rows_v[i, pl.ds(16, 16)] = r1 * scale
                rows_v[i, pl.ds(32, 16)] = r2 * scale
                rows_v[i, pl.ds(48, 16)] = r3 * scale
                return c

            lax.fori_loop(0, CHUNK, row_fix, 0, unroll=4)

            pltpu.sync_copy(rows_v, out_hbm.at[pl.ds(row_off, CHUNK)])
            return carry

        lax.fori_loop(0, steps, step, 0)

    return body(idx2d, table)


def kernel(token_ids, embeddings):
    bsz, seq = token_ids.shape
    n_rows = bsz * seq
    idx2d = token_ids.reshape(n_rows // GATHER, GATHER).astype(jnp.int32)
    out = _sc_embed_logmap(idx2d, embeddings, n_rows)
    return out.reshape(bsz, seq, D_MODEL)
